# baseline (speedup 1.0000x reference)
import jax
import jax.numpy as jnp
from jax import lax
from jax.experimental import pallas as pl
from jax.experimental.pallas import tpu as pltpu
from jax.experimental.pallas import tpu_sc as plsc

N = 100000
D = 32
SIZES = (120, 17, 13, 5, 10, 3, 7)
NT = len(SIZES)

NC = 2
NS = 16
NW = NC * NS
L = 16

NPAD = 102400
PER_TILE = NPAD // NW
CHUNK = 800
NCHUNKS = PER_TILE // CHUNK

G1_ROWS = 120 * 3
G2_ROWS = 13 * 10
U_ROWS = 17 * 5
G3_ROWS = U_ROWS * 7


def _row(ref, r):
    return (ref[pl.ds(r * D, L)], ref[pl.ds(r * D + L, L)])


def _sc_body(*refs):
    idx_hbm = refs[0:NT]
    w_hbm = refs[NT:2 * NT]
    out_hbm = refs[2 * NT]
    sc = refs[2 * NT + 1:]
    w_v = sc[0:NT]
    g1_v, g2_v, u_v, g3_v = sc[NT:NT + 4]
    idx_v = sc[NT + 4:2 * NT + 4]
    out_v = sc[2 * NT + 4]

    c = lax.axis_index("c")
    s = lax.axis_index("s")
    wid = s * NC + c
    base = wid * PER_TILE

    for t in range(NT):
        pltpu.sync_copy(w_hbm[t], w_v[t])

    def _combine(dst, a_ref, b_rows_static, nb):
        def body(i, carry):
            alo, ahi = _row(a_ref, i)
            for r in range(nb):
                row = (i * nb + r) * D
                dst[pl.ds(row, L)] = alo + b_rows_static[r][0]
                dst[pl.ds(row + L, L)] = ahi + b_rows_static[r][1]
            return carry
        return body

    ar_rows = [_row(w_v[5], r) for r in range(3)]
    lax.fori_loop(0, 120, _combine(g1_v, w_v[0], ar_rows, 3), 0)
    nh_rows = [_row(w_v[4], r) for r in range(10)]
    lax.fori_loop(0, 13, _combine(g2_v, w_v[2], nh_rows, 10), 0)
    ct_rows = [_row(w_v[3], r) for r in range(5)]
    lax.fori_loop(0, 17, _combine(u_v, w_v[1], ct_rows, 5), 0)
    hy_rows = [_row(w_v[6], r) for r in range(7)]
    lax.fori_loop(0, U_ROWS, _combine(g3_v, u_v, hy_rows, 7), 0)

    iota_d = lax.iota(jnp.int32, L) * D

    def chunk_body(ci, carry):
        row0 = base + ci * CHUNK
        for t in range(NT):
            pltpu.sync_copy(idx_hbm[t].at[pl.ds(row0, CHUNK)], idx_v[t])

        def group_body(g, carry2):
            a0 = g * L
            v = [idx_v[t][pl.ds(a0, L)] for t in range(NT)]
            f1 = v[0] * (3 * D) + v[5] * D
            f2 = v[2] * (10 * D) + v[4] * D
            f3 = v[1] * (35 * D) + v[3] * (7 * D) + v[6] * D
            obase = iota_d + a0 * D
            for d in range(D):
                acc = plsc.load_gather(g1_v, [f1 + d])
                acc = acc + plsc.load_gather(g2_v, [f2 + d])
                acc = acc + plsc.load_gather(g3_v, [f3 + d])
                plsc.store_scatter(out_v, [obase + d], acc)
            return carry2

        lax.fori_loop(0, CHUNK // L, group_body, 0)
        pltpu.sync_copy(out_v, out_hbm.at[pl.ds(row0 * D, CHUNK * D)])
        return carry

    lax.fori_loop(0, NCHUNKS, chunk_body, 0)


@jax.jit
def _run(idxs, tables_flat):
    mesh = plsc.VectorSubcoreMesh(
        core_axis_name="c", subcore_axis_name="s",
        num_cores=NC, num_subcores=NS)
    scratch = (
        [pltpu.VMEM((SIZES[t] * D,), jnp.float32) for t in range(NT)]
        + [pltpu.VMEM((G1_ROWS * D,), jnp.float32),
           pltpu.VMEM((G2_ROWS * D,), jnp.float32),
           pltpu.VMEM((U_ROWS * D,), jnp.float32),
           pltpu.VMEM((G3_ROWS * D,), jnp.float32)]
        + [pltpu.VMEM((CHUNK,), jnp.int32) for _ in range(NT)]
        + [pltpu.VMEM((CHUNK * D,), jnp.float32)]
    )
    fn = pl.kernel(
        _sc_body,
        out_type=jax.ShapeDtypeStruct((NPAD * D,), jnp.float32),
        mesh=mesh,
        scratch_types=scratch,
        compiler_params=pltpu.CompilerParams(needs_layout_passes=False),
    )
    return fn(*idxs, *tables_flat)


def kernel(atomic_num, formal_charge, degree, chiral_tag, total_numHs,
           is_aromatic, hybridization, W_atomic_num, W_formal_charge,
           W_degree, W_chiral_tag, W_total_numHs, W_is_aromatic,
           W_hybridization):
    idxs = [atomic_num, formal_charge, degree, chiral_tag, total_numHs,
            is_aromatic, hybridization]
    tables = [W_atomic_num, W_formal_charge, W_degree, W_chiral_tag,
              W_total_numHs, W_is_aromatic, W_hybridization]
    pad = NPAD - N
    idxs = [jnp.concatenate([i, jnp.zeros((pad,), jnp.int32)]) for i in idxs]
    tables_flat = [w.reshape(-1) for w in tables]
    out = _run(idxs, tables_flat)
    return out.reshape(NPAD, D)[:N]


# speedup vs baseline: 2.3738x; 2.3738x over previous
import jax
import jax.numpy as jnp
from jax import lax
from jax.experimental import pallas as pl
from jax.experimental.pallas import tpu as pltpu
from jax.experimental.pallas import tpu_sc as plsc

N = 100000
D = 32
SIZES = (120, 17, 13, 5, 10, 3, 7)
NT = len(SIZES)

NC = 2
NS = 16
NW = NC * NS
L = 16

NPAD = 102400
PER_TILE = NPAD // NW
CHUNK = 800
NCHUNKS = PER_TILE // CHUNK

G1_ROWS = 120 * 3
G2_ROWS = 13 * 10
U_ROWS = 17 * 5
G3_ROWS = U_ROWS * 7


def _row(ref, r):
    return (ref[pl.ds(r * D, L)], ref[pl.ds(r * D + L, L)])


def _sc_body(*refs):
    idx_hbm = refs[0:NT]
    w_hbm = refs[NT:2 * NT]
    out_hbm = refs[2 * NT]
    sc = refs[2 * NT + 1:]
    w_v = sc[0:NT]
    g1_v, g2_v, u_v, g3_v = sc[NT:NT + 4]
    idx_v = sc[NT + 4:2 * NT + 4]
    out_v = sc[2 * NT + 4]

    c = lax.axis_index("c")
    s = lax.axis_index("s")
    wid = s * NC + c
    base = wid * PER_TILE

    for t in range(NT):
        pltpu.sync_copy(w_hbm[t], w_v[t])

    def _combine(dst, a_ref, b_rows_static, nb):
        def body(i, carry):
            alo, ahi = _row(a_ref, i)
            for r in range(nb):
                row = (i * nb + r) * D
                dst[pl.ds(row, L)] = alo + b_rows_static[r][0]
                dst[pl.ds(row + L, L)] = ahi + b_rows_static[r][1]
            return carry
        return body

    ar_rows = [_row(w_v[5], r) for r in range(3)]
    lax.fori_loop(0, 120, _combine(g1_v, w_v[0], ar_rows, 3), 0)
    nh_rows = [_row(w_v[4], r) for r in range(10)]
    lax.fori_loop(0, 13, _combine(g2_v, w_v[2], nh_rows, 10), 0)
    ct_rows = [_row(w_v[3], r) for r in range(5)]
    lax.fori_loop(0, 17, _combine(u_v, w_v[1], ct_rows, 5), 0)
    hy_rows = [_row(w_v[6], r) for r in range(7)]
    lax.fori_loop(0, U_ROWS, _combine(g3_v, u_v, hy_rows, 7), 0)

    iota = lax.iota(jnp.int32, L)
    iota_hi = iota + L

    def chunk_body(ci, carry):
        row0 = base + ci * CHUNK
        for t in range(NT):
            pltpu.sync_copy(idx_hbm[t].at[pl.ds(row0, CHUNK)], idx_v[t])

        def group_body(g, carry2):
            a0 = g * L
            v = [idx_v[t][pl.ds(a0, L)] for t in range(NT)]
            f1 = v[0] * (3 * D) + v[5] * D
            f2 = v[2] * (10 * D) + v[4] * D
            f3 = v[1] * (35 * D) + v[3] * (7 * D) + v[6] * D
            for j in range(L):
                jsplat = jnp.full((L,), j, jnp.int32)
                b1 = f1.at[jsplat].get(mode="promise_in_bounds")
                b2 = f2.at[jsplat].get(mode="promise_in_bounds")
                b3 = f3.at[jsplat].get(mode="promise_in_bounds")
                acc_lo = plsc.load_gather(g1_v, [b1 + iota])
                acc_hi = plsc.load_gather(g1_v, [b1 + iota_hi])
                acc_lo = acc_lo + plsc.load_gather(g2_v, [b2 + iota])
                acc_hi = acc_hi + plsc.load_gather(g2_v, [b2 + iota_hi])
                acc_lo = acc_lo + plsc.load_gather(g3_v, [b3 + iota])
                acc_hi = acc_hi + plsc.load_gather(g3_v, [b3 + iota_hi])
                o = (a0 + j) * D
                out_v[pl.ds(o, L)] = acc_lo
                out_v[pl.ds(o + L, L)] = acc_hi
            return carry2

        lax.fori_loop(0, CHUNK // L, group_body, 0)
        pltpu.sync_copy(out_v, out_hbm.at[pl.ds(row0 * D, CHUNK * D)])
        return carry

    lax.fori_loop(0, NCHUNKS, chunk_body, 0)


@jax.jit
def _run(idxs, tables_flat):
    mesh = plsc.VectorSubcoreMesh(
        core_axis_name="c", subcore_axis_name="s",
        num_cores=NC, num_subcores=NS)
    scratch = (
        [pltpu.VMEM((SIZES[t] * D,), jnp.float32) for t in range(NT)]
        + [pltpu.VMEM((G1_ROWS * D,), jnp.float32),
           pltpu.VMEM((G2_ROWS * D,), jnp.float32),
           pltpu.VMEM((U_ROWS * D,), jnp.float32),
           pltpu.VMEM((G3_ROWS * D,), jnp.float32)]
        + [pltpu.VMEM((CHUNK,), jnp.int32) for _ in range(NT)]
        + [pltpu.VMEM((CHUNK * D,), jnp.float32)]
    )
    fn = pl.kernel(
        _sc_body,
        out_type=jax.ShapeDtypeStruct((NPAD * D,), jnp.float32),
        mesh=mesh,
        scratch_types=scratch,
        compiler_params=pltpu.CompilerParams(needs_layout_passes=False),
    )
    return fn(*idxs, *tables_flat)


def kernel(atomic_num, formal_charge, degree, chiral_tag, total_numHs,
           is_aromatic, hybridization, W_atomic_num, W_formal_charge,
           W_degree, W_chiral_tag, W_total_numHs, W_is_aromatic,
           W_hybridization):
    idxs = [atomic_num, formal_charge, degree, chiral_tag, total_numHs,
            is_aromatic, hybridization]
    tables = [W_atomic_num, W_formal_charge, W_degree, W_chiral_tag,
              W_total_numHs, W_is_aromatic, W_hybridization]
    pad = NPAD - N
    idxs = [jnp.concatenate([i, jnp.zeros((pad,), jnp.int32)]) for i in idxs]
    tables_flat = [w.reshape(-1) for w in tables]
    out = _run(idxs, tables_flat)
    return out.reshape(NPAD, D)[:N]


# speedup vs baseline: 2.7844x; 1.1729x over previous
import jax
import jax.numpy as jnp
from jax import lax
from jax.experimental import pallas as pl
from jax.experimental.pallas import tpu as pltpu
from jax.experimental.pallas import tpu_sc as plsc

N = 100000
D = 32
SIZES = (120, 17, 13, 5, 10, 3, 7)
NT = len(SIZES)

NC = 2
NS = 16
NW = NC * NS
L = 16

PER_TILE = 3200
CHUNK = 800
NCHUNKS = PER_TILE // CHUNK
LAST_START = N - CHUNK

G1_ROWS = 120 * 3
G2_ROWS = 13 * 10
U_ROWS = 17 * 5
G3_ROWS = U_ROWS * 7


def _row(ref, r):
    return (ref[pl.ds(r * D, L)], ref[pl.ds(r * D + L, L)])


def _sc_body(*refs):
    idx_hbm = refs[0:NT]
    w_hbm = refs[NT:2 * NT]
    out_hbm = refs[2 * NT]
    sc = refs[2 * NT + 1:]
    w_v = sc[0:NT]
    g1_v, g2_v, u_v, g3_v = sc[NT:NT + 4]
    idx_v = sc[NT + 4:2 * NT + 4]
    out_v = sc[2 * NT + 4]

    c = lax.axis_index("c")
    s = lax.axis_index("s")
    wid = s * NC + c
    base = wid * PER_TILE

    for t in range(NT):
        pltpu.sync_copy(w_hbm[t], w_v[t])

    def _combine(dst, a_ref, b_rows_static, nb):
        def body(i, carry):
            alo, ahi = _row(a_ref, i)
            for r in range(nb):
                row = (i * nb + r) * D
                dst[pl.ds(row, L)] = alo + b_rows_static[r][0]
                dst[pl.ds(row + L, L)] = ahi + b_rows_static[r][1]
            return carry
        return body

    ar_rows = [_row(w_v[5], r) for r in range(3)]
    lax.fori_loop(0, 120, _combine(g1_v, w_v[0], ar_rows, 3), 0)
    nh_rows = [_row(w_v[4], r) for r in range(10)]
    lax.fori_loop(0, 13, _combine(g2_v, w_v[2], nh_rows, 10), 0)
    ct_rows = [_row(w_v[3], r) for r in range(5)]
    lax.fori_loop(0, 17, _combine(u_v, w_v[1], ct_rows, 5), 0)
    hy_rows = [_row(w_v[6], r) for r in range(7)]
    lax.fori_loop(0, U_ROWS, _combine(g3_v, u_v, hy_rows, 7), 0)

    iota = lax.iota(jnp.int32, L)
    iota_hi = iota + L

    def chunk_body(ci, carry):
        row0 = pl.multiple_of(
            jnp.minimum(base + ci * CHUNK, LAST_START), 8)
        for t in range(NT):
            pltpu.sync_copy(idx_hbm[t].at[pl.ds(row0, CHUNK)], idx_v[t])

        def group_body(g, carry2):
            a0 = g * L
            v = [idx_v[t][pl.ds(a0, L)] for t in range(NT)]
            f1 = v[0] * (3 * D) + v[5] * D
            f2 = v[2] * (10 * D) + v[4] * D
            f3 = v[1] * (35 * D) + v[3] * (7 * D) + v[6] * D
            for j in range(L):
                jsplat = jnp.full((L,), j, jnp.int32)
                b1 = f1.at[jsplat].get(mode="promise_in_bounds")
                b2 = f2.at[jsplat].get(mode="promise_in_bounds")
                b3 = f3.at[jsplat].get(mode="promise_in_bounds")
                acc_lo = plsc.load_gather(g1_v, [b1 + iota])
                acc_hi = plsc.load_gather(g1_v, [b1 + iota_hi])
                acc_lo = acc_lo + plsc.load_gather(g2_v, [b2 + iota])
                acc_hi = acc_hi + plsc.load_gather(g2_v, [b2 + iota_hi])
                acc_lo = acc_lo + plsc.load_gather(g3_v, [b3 + iota])
                acc_hi = acc_hi + plsc.load_gather(g3_v, [b3 + iota_hi])
                o = (a0 + j) * D
                out_v[pl.ds(o, L)] = acc_lo
                out_v[pl.ds(o + L, L)] = acc_hi
            return carry2

        lax.fori_loop(0, CHUNK // L, group_body, 0)
        pltpu.sync_copy(
            out_v, out_hbm.at[pl.ds(pl.multiple_of(row0 * D, 256), CHUNK * D)])
        return carry

    lax.fori_loop(0, NCHUNKS, chunk_body, 0)


@jax.jit
def _run(idxs, tables_flat):
    mesh = plsc.VectorSubcoreMesh(
        core_axis_name="c", subcore_axis_name="s",
        num_cores=NC, num_subcores=NS)
    scratch = (
        [pltpu.VMEM((SIZES[t] * D,), jnp.float32) for t in range(NT)]
        + [pltpu.VMEM((G1_ROWS * D,), jnp.float32),
           pltpu.VMEM((G2_ROWS * D,), jnp.float32),
           pltpu.VMEM((U_ROWS * D,), jnp.float32),
           pltpu.VMEM((G3_ROWS * D,), jnp.float32)]
        + [pltpu.VMEM((CHUNK,), jnp.int32) for _ in range(NT)]
        + [pltpu.VMEM((CHUNK * D,), jnp.float32)]
    )
    fn = pl.kernel(
        _sc_body,
        out_type=jax.ShapeDtypeStruct((N * D,), jnp.float32),
        mesh=mesh,
        scratch_types=scratch,
        compiler_params=pltpu.CompilerParams(needs_layout_passes=False),
    )
    return fn(*idxs, *tables_flat)


def kernel(atomic_num, formal_charge, degree, chiral_tag, total_numHs,
           is_aromatic, hybridization, W_atomic_num, W_formal_charge,
           W_degree, W_chiral_tag, W_total_numHs, W_is_aromatic,
           W_hybridization):
    idxs = [atomic_num, formal_charge, degree, chiral_tag, total_numHs,
            is_aromatic, hybridization]
    tables = [W_atomic_num, W_formal_charge, W_degree, W_chiral_tag,
              W_total_numHs, W_is_aromatic, W_hybridization]
    tables_flat = [w.reshape(-1) for w in tables]
    out = _run(idxs, tables_flat)
    return out.reshape(N, D)


# speedup vs baseline: 3.1134x; 1.1182x over previous
import jax
import jax.numpy as jnp
from jax import lax
from jax.experimental import pallas as pl
from jax.experimental.pallas import tpu as pltpu
from jax.experimental.pallas import tpu_sc as plsc

N = 100000
D = 32
SIZES = (120, 17, 13, 5, 10, 3, 7)
NT = len(SIZES)

NC = 2
NS = 16
NW = NC * NS
L = 16

PER_TILE = 3200
CHUNK = 800
NCHUNKS = PER_TILE // CHUNK
LAST_START = N - CHUNK

G1_ROWS = 120 * 3
G2_ROWS = 13 * 10
U_ROWS = 17 * 5
G3_ROWS = U_ROWS * 7


def _row(ref, r):
    return (ref[pl.ds(r * D, L)], ref[pl.ds(r * D + L, L)])


def _sc_body(*refs):
    idx_hbm = refs[0:NT]
    w_hbm = refs[NT:2 * NT]
    out_hbm = refs[2 * NT]
    sc = refs[2 * NT + 1:]
    w_v = sc[0:NT]
    g1_v, g2_v, u_v, g3_v = sc[NT:NT + 4]
    idx_v = (sc[NT + 4:2 * NT + 4], sc[2 * NT + 4:3 * NT + 4])
    out_v = sc[3 * NT + 4:3 * NT + 6]
    idx_sem = sc[3 * NT + 6:3 * NT + 8]
    out_sem = sc[3 * NT + 8:3 * NT + 10]

    c = lax.axis_index("c")
    s = lax.axis_index("s")
    wid = s * NC + c
    base = wid * PER_TILE

    for t in range(NT):
        pltpu.sync_copy(w_hbm[t], w_v[t])

    def _pack_row(lo, hi):
        return plsc.bitcast(
            plsc.pack(lo, hi, format=plsc.PackFormat.INTERLEAVED), jnp.int32)

    def _combine_packed(dst, a_ref, b_rows_static, nb):
        def body(i, carry):
            alo, ahi = _row(a_ref, i)
            for r in range(nb):
                row = (i * nb + r) * L
                dst[pl.ds(row, L)] = _pack_row(
                    alo + b_rows_static[r][0], ahi + b_rows_static[r][1])
            return carry
        return body

    def _combine_f32(dst, a_ref, b_rows_static, nb):
        def body(i, carry):
            alo, ahi = _row(a_ref, i)
            for r in range(nb):
                row = (i * nb + r) * D
                dst[pl.ds(row, L)] = alo + b_rows_static[r][0]
                dst[pl.ds(row + L, L)] = ahi + b_rows_static[r][1]
            return carry
        return body

    ar_rows = [_row(w_v[5], r) for r in range(3)]
    lax.fori_loop(0, 120, _combine_packed(g1_v, w_v[0], ar_rows, 3), 0)
    nh_rows = [_row(w_v[4], r) for r in range(10)]
    lax.fori_loop(0, 13, _combine_packed(g2_v, w_v[2], nh_rows, 10), 0)
    ct_rows = [_row(w_v[3], r) for r in range(5)]
    lax.fori_loop(0, 17, _combine_f32(u_v, w_v[1], ct_rows, 5), 0)
    hy_rows = [_row(w_v[6], r) for r in range(7)]
    lax.fori_loop(0, U_ROWS, _combine_packed(g3_v, u_v, hy_rows, 7), 0)

    iota = lax.iota(jnp.int32, L)
    iota_hi = iota + L

    def chunk_start(ci):
        return pl.multiple_of(jnp.minimum(base + ci * CHUNK, LAST_START), 8)

    def fire_idx(ci, buf):
        row0 = chunk_start(ci)
        return [
            pltpu.async_copy(
                idx_hbm[t].at[pl.ds(row0, CHUNK)], idx_v[buf][t], idx_sem[buf])
            for t in range(NT)
        ]

    idx_inflight = {0: fire_idx(0, 0)}
    out_inflight = {}

    for ci in range(NCHUNKS):
        cur = ci % 2
        if ci + 1 < NCHUNKS:
            idx_inflight[ci + 1] = fire_idx(ci + 1, 1 - cur)
        for cp in idx_inflight.pop(ci):
            cp.wait()
        if ci - 2 in out_inflight:
            out_inflight.pop(ci - 2).wait()

        def group_body(g, carry2, cur=cur):
            a0 = g * L
            v = [idx_v[cur][t][pl.ds(a0, L)] for t in range(NT)]
            f1 = v[0] * (3 * L) + v[5] * L
            f2 = v[2] * (10 * L) + v[4] * L
            f3 = (v[1] * 35 + v[3] * 7 + v[6]) * L
            for j in range(L):
                jsplat = jnp.full((L,), j, jnp.int32)
                b1 = f1.at[jsplat].get(mode="promise_in_bounds")
                b2 = f2.at[jsplat].get(mode="promise_in_bounds")
                b3 = f3.at[jsplat].get(mode="promise_in_bounds")
                r1 = plsc.bitcast(
                    plsc.load_gather(g1_v, [b1 + iota]), jnp.bfloat16)
                r2 = plsc.bitcast(
                    plsc.load_gather(g2_v, [b2 + iota]), jnp.bfloat16)
                r3 = plsc.bitcast(
                    plsc.load_gather(g3_v, [b3 + iota]), jnp.bfloat16)
                acc_lo, acc_hi = plsc.unpack(
                    r1 + r2 + r3, format=plsc.PackFormat.INTERLEAVED)
                o = (a0 + j) * D
                out_v[cur][pl.ds(o, L)] = acc_lo
                out_v[cur][pl.ds(o + L, L)] = acc_hi
            return carry2

        lax.fori_loop(0, CHUNK // L, group_body, 0)
        row0 = chunk_start(ci)
        out_inflight[ci] = pltpu.async_copy(
            out_v[cur],
            out_hbm.at[pl.ds(pl.multiple_of(row0 * D, 256), CHUNK * D)],
            out_sem[cur])

    for cp in out_inflight.values():
        cp.wait()


@jax.jit
def _run(idxs, tables_flat):
    mesh = plsc.VectorSubcoreMesh(
        core_axis_name="c", subcore_axis_name="s",
        num_cores=NC, num_subcores=NS)
    scratch = (
        [pltpu.VMEM((SIZES[t] * D,), jnp.float32) for t in range(NT)]
        + [pltpu.VMEM((G1_ROWS * L,), jnp.int32),
           pltpu.VMEM((G2_ROWS * L,), jnp.int32),
           pltpu.VMEM((U_ROWS * D,), jnp.float32),
           pltpu.VMEM((G3_ROWS * L,), jnp.int32)]
        + [pltpu.VMEM((CHUNK,), jnp.int32) for _ in range(2 * NT)]
        + [pltpu.VMEM((CHUNK * D,), jnp.float32) for _ in range(2)]
        + [pltpu.SemaphoreType.DMA for _ in range(4)]
    )
    fn = pl.kernel(
        _sc_body,
        out_type=jax.ShapeDtypeStruct((N * D,), jnp.float32),
        mesh=mesh,
        scratch_types=scratch,
        compiler_params=pltpu.CompilerParams(needs_layout_passes=False),
    )
    return fn(*idxs, *tables_flat)


def kernel(atomic_num, formal_charge, degree, chiral_tag, total_numHs,
           is_aromatic, hybridization, W_atomic_num, W_formal_charge,
           W_degree, W_chiral_tag, W_total_numHs, W_is_aromatic,
           W_hybridization):
    idxs = [atomic_num, formal_charge, degree, chiral_tag, total_numHs,
            is_aromatic, hybridization]
    tables = [W_atomic_num, W_formal_charge, W_degree, W_chiral_tag,
              W_total_numHs, W_is_aromatic, W_hybridization]
    tables_flat = [w.reshape(-1) for w in tables]
    out = _run(idxs, tables_flat)
    return out.reshape(N, D)


# speedup vs baseline: 5.2935x; 1.7002x over previous
import jax
import jax.numpy as jnp
from jax import lax
from jax.experimental import pallas as pl
from jax.experimental.pallas import tpu as pltpu
from jax.experimental.pallas import tpu_sc as plsc

N = 100000
D = 32
SIZES = (120, 17, 13, 5, 10, 3, 7)
NT = len(SIZES)

NC = 2
NS = 16
NW = NC * NS
L = 16

PER_TILE = 3200
CHUNK = 640
NCHUNKS = PER_TILE // CHUNK
LAST_START = 100096 - CHUNK

RSTRIDE = 17
G1_ROWS = 120 * 3
G2_ROWS = 13 * 10
U_ROWS = 17 * 5
G3_ROWS = U_ROWS * 7


def _row(ref, r):
    return (ref[pl.ds(r * D, L)], ref[pl.ds(r * D + L, L)])


def _sc_body(*refs):
    idx_hbm = refs[0:NT]
    w_hbm = refs[NT:2 * NT]
    out_hbm = refs[2 * NT]
    sc = refs[2 * NT + 1:]
    w_v = sc[0:NT]
    g1_v, g2_v, u_v, g3_v = sc[NT:NT + 4]
    idx_v = (sc[NT + 4:2 * NT + 4], sc[2 * NT + 4:3 * NT + 4])
    out_v = sc[3 * NT + 4:3 * NT + 6]
    idx_sem = sc[3 * NT + 6:3 * NT + 8]
    out_sem = sc[3 * NT + 8:3 * NT + 10]

    c = lax.axis_index("c")
    s = lax.axis_index("s")
    wid = s * NC + c
    base = wid * PER_TILE

    iota = lax.iota(jnp.int32, L)

    for t in range(NT):
        pltpu.sync_copy(w_hbm[t], w_v[t])

    def _pack_row(lo, hi):
        return plsc.bitcast(
            plsc.pack(lo, hi, format=plsc.PackFormat.INTERLEAVED), jnp.int32)

    def _combine_packed(dst, a_ref, b_rows_static, nb):
        def body(i, carry):
            alo, ahi = _row(a_ref, i)
            for r in range(nb):
                row = (i * nb + r) * RSTRIDE
                plsc.store_scatter(
                    dst, [iota + row],
                    _pack_row(alo + b_rows_static[r][0],
                              ahi + b_rows_static[r][1]))
            return carry
        return body

    def _combine_f32(dst, a_ref, b_rows_static, nb):
        def body(i, carry):
            alo, ahi = _row(a_ref, i)
            for r in range(nb):
                row = (i * nb + r) * D
                dst[pl.ds(row, L)] = alo + b_rows_static[r][0]
                dst[pl.ds(row + L, L)] = ahi + b_rows_static[r][1]
            return carry
        return body

    ar_rows = [_row(w_v[5], r) for r in range(3)]
    lax.fori_loop(0, 120, _combine_packed(g1_v, w_v[0], ar_rows, 3), 0)
    nh_rows = [_row(w_v[4], r) for r in range(10)]
    lax.fori_loop(0, 13, _combine_packed(g2_v, w_v[2], nh_rows, 10), 0)
    ct_rows = [_row(w_v[3], r) for r in range(5)]
    lax.fori_loop(0, 17, _combine_f32(u_v, w_v[1], ct_rows, 5), 0)
    hy_rows = [_row(w_v[6], r) for r in range(7)]
    lax.fori_loop(0, U_ROWS, _combine_packed(g3_v, u_v, hy_rows, 7), 0)

    def chunk_start(ci):
        return pl.multiple_of(jnp.minimum(base + ci * CHUNK, LAST_START), 8)

    def fire_idx(ci, buf):
        row0 = chunk_start(ci)
        return [
            pltpu.async_copy(
                idx_hbm[t].at[pl.ds(row0, CHUNK)], idx_v[buf][t], idx_sem[buf])
            for t in range(NT)
        ]

    def fire_out(ci, buf):
        row0 = chunk_start(ci)
        return [
            pltpu.async_copy(
                out_v[buf].at[pl.ds(d * CHUNK, CHUNK)],
                out_hbm.at[d, pl.ds(row0, CHUNK)], out_sem[buf])
            for d in range(D)
        ]

    idx_inflight = {0: fire_idx(0, 0)}
    out_inflight = {}

    for ci in range(NCHUNKS):
        cur = ci % 2
        if ci + 1 < NCHUNKS:
            idx_inflight[ci + 1] = fire_idx(ci + 1, 1 - cur)
        for cp in idx_inflight.pop(ci):
            cp.wait()
        if ci - 2 in out_inflight:
            for cp in out_inflight.pop(ci - 2):
                cp.wait()

        def group_body(g, carry2, cur=cur):
            a0 = g * L
            v = [idx_v[cur][t][pl.ds(a0, L)] for t in range(NT)]
            f1 = (v[0] * 3 + v[5]) * RSTRIDE
            f2 = (v[2] * 10 + v[4]) * RSTRIDE
            f3 = (v[1] * 35 + v[3] * 7 + v[6]) * RSTRIDE
            zero = jnp.zeros((L,), jnp.int32)
            f1 = lax.max(lax.min(f1, (G1_ROWS - 1) * RSTRIDE), zero)
            f2 = lax.max(lax.min(f2, (G2_ROWS - 1) * RSTRIDE), zero)
            f3 = lax.max(lax.min(f3, (G3_ROWS - 1) * RSTRIDE), zero)
            for k in range(L):
                r1 = plsc.bitcast(
                    plsc.load_gather(g1_v, [f1 + k]), jnp.bfloat16)
                r2 = plsc.bitcast(
                    plsc.load_gather(g2_v, [f2 + k]), jnp.bfloat16)
                r3 = plsc.bitcast(
                    plsc.load_gather(g3_v, [f3 + k]), jnp.bfloat16)
                acc_k, acc_khi = plsc.unpack(
                    r1 + r2 + r3, format=plsc.PackFormat.INTERLEAVED)
                out_v[cur][pl.ds(k * CHUNK + a0, L)] = acc_k
                out_v[cur][pl.ds((k + L) * CHUNK + a0, L)] = acc_khi
            return carry2

        lax.fori_loop(0, CHUNK // L, group_body, 0)
        out_inflight[ci] = fire_out(ci, cur)

    for cps in out_inflight.values():
        for cp in cps:
            cp.wait()


@jax.jit
def _run(idxs, tables_flat):
    mesh = plsc.VectorSubcoreMesh(
        core_axis_name="c", subcore_axis_name="s",
        num_cores=NC, num_subcores=NS)
    scratch = (
        [pltpu.VMEM((SIZES[t] * D,), jnp.float32) for t in range(NT)]
        + [pltpu.VMEM((G1_ROWS * RSTRIDE,), jnp.int32),
           pltpu.VMEM((G2_ROWS * RSTRIDE,), jnp.int32),
           pltpu.VMEM((U_ROWS * D,), jnp.float32),
           pltpu.VMEM((G3_ROWS * RSTRIDE,), jnp.int32)]
        + [pltpu.VMEM((CHUNK,), jnp.int32) for _ in range(2 * NT)]
        + [pltpu.VMEM((CHUNK * D,), jnp.float32) for _ in range(2)]
        + [pltpu.SemaphoreType.DMA for _ in range(4)]
    )
    fn = pl.kernel(
        _sc_body,
        out_type=jax.ShapeDtypeStruct((D, N), jnp.float32),
        mesh=mesh,
        scratch_types=scratch,
        compiler_params=pltpu.CompilerParams(needs_layout_passes=False),
    )
    return fn(*idxs, *tables_flat)


def kernel(atomic_num, formal_charge, degree, chiral_tag, total_numHs,
           is_aromatic, hybridization, W_atomic_num, W_formal_charge,
           W_degree, W_chiral_tag, W_total_numHs, W_is_aromatic,
           W_hybridization):
    idxs = [atomic_num, formal_charge, degree, chiral_tag, total_numHs,
            is_aromatic, hybridization]
    tables = [W_atomic_num, W_formal_charge, W_degree, W_chiral_tag,
              W_total_numHs, W_is_aromatic, W_hybridization]
    tables_flat = [w.reshape(-1) for w in tables]
    out = _run(idxs, tables_flat)
    return out.T


# speedup vs baseline: 5.5827x; 1.0546x over previous
import jax
import jax.numpy as jnp
from jax import lax
from jax.experimental import pallas as pl
from jax.experimental.pallas import tpu as pltpu
from jax.experimental.pallas import tpu_sc as plsc

N = 100000
D = 32
SIZES = (120, 17, 13, 5, 10, 3, 7)
NT = len(SIZES)

NC = 2
NS = 16
NW = NC * NS
L = 16

PER_TILE = 3200
CHUNK = 640
NCHUNKS = PER_TILE // CHUNK
LAST_START = 100096 - CHUNK

RSTRIDE = 17
G1_ROWS = 120 * 3
G2_ROWS = 13 * 10
U_ROWS = 17 * 5
G3_ROWS = U_ROWS * 7


def _row(ref, r):
    return (ref[pl.ds(r * D, L)], ref[pl.ds(r * D + L, L)])


def _sc_body(*refs):
    idx_hbm = refs[0:NT]
    w_hbm = refs[NT:2 * NT]
    out_hbm = refs[2 * NT]
    sc = refs[2 * NT + 1:]
    w_v = sc[0:NT]
    g1_v, g2_v, u_v, g3_v = sc[NT:NT + 4]
    idx_v = (sc[NT + 4:2 * NT + 4], sc[2 * NT + 4:3 * NT + 4])
    out_v = sc[3 * NT + 4:3 * NT + 6]
    idx_sem = sc[3 * NT + 6:3 * NT + 8]
    out_sem = sc[3 * NT + 8:3 * NT + 10]

    c = lax.axis_index("c")
    s = lax.axis_index("s")
    wid = s * NC + c
    base = wid * PER_TILE

    iota = lax.iota(jnp.int32, L)

    for t in range(NT):
        pltpu.sync_copy(w_hbm[t], w_v[t])

    def _pack_row(lo, hi):
        return plsc.bitcast(
            plsc.pack(lo, hi, format=plsc.PackFormat.INTERLEAVED), jnp.int32)

    def _combine_packed(dst, a_ref, b_rows_static, nb):
        def body(i, carry):
            alo, ahi = _row(a_ref, i)
            for r in range(nb):
                row = (i * nb + r) * RSTRIDE
                plsc.store_scatter(
                    dst, [iota + row],
                    _pack_row(alo + b_rows_static[r][0],
                              ahi + b_rows_static[r][1]))
            return carry
        return body

    def _combine_f32(dst, a_ref, b_rows_static, nb):
        def body(i, carry):
            alo, ahi = _row(a_ref, i)
            for r in range(nb):
                row = (i * nb + r) * D
                dst[pl.ds(row, L)] = alo + b_rows_static[r][0]
                dst[pl.ds(row + L, L)] = ahi + b_rows_static[r][1]
            return carry
        return body

    ar_rows = [_row(w_v[5], r) for r in range(3)]
    lax.fori_loop(0, 120, _combine_packed(g1_v, w_v[0], ar_rows, 3), 0)
    nh_rows = [_row(w_v[4], r) for r in range(10)]
    lax.fori_loop(0, 13, _combine_packed(g2_v, w_v[2], nh_rows, 10), 0)
    ct_rows = [_row(w_v[3], r) for r in range(5)]
    lax.fori_loop(0, 17, _combine_f32(u_v, w_v[1], ct_rows, 5), 0)
    hy_rows = [_row(w_v[6], r) for r in range(7)]
    lax.fori_loop(0, U_ROWS, _combine_packed(g3_v, u_v, hy_rows, 7), 0)

    def chunk_start(ci):
        return pl.multiple_of(jnp.minimum(base + ci * CHUNK, LAST_START), 8)

    def fire_idx(ci, buf):
        row0 = chunk_start(ci)
        return [
            pltpu.async_copy(
                idx_hbm[t].at[pl.ds(row0, CHUNK)], idx_v[buf][t], idx_sem[buf])
            for t in range(NT)
        ]

    def fire_out(ci, buf):
        row0 = chunk_start(ci)
        return [
            pltpu.async_copy(
                out_v[buf].at[pl.ds(d * CHUNK, CHUNK)],
                out_hbm.at[d, pl.ds(row0, CHUNK)], out_sem[buf])
            for d in range(D)
        ]

    idx_inflight = {0: fire_idx(0, 0)}
    out_inflight = {}

    for ci in range(NCHUNKS):
        cur = ci % 2
        if ci + 1 < NCHUNKS:
            idx_inflight[ci + 1] = fire_idx(ci + 1, 1 - cur)
        for cp in idx_inflight.pop(ci):
            cp.wait()
        if ci - 2 in out_inflight:
            for cp in out_inflight.pop(ci - 2):
                cp.wait()

        @plsc.parallel_loop(0, CHUNK // L, 1, unroll=2)
        def group_body(g, cur=cur):
            a0 = g * L
            v = [idx_v[cur][t][pl.ds(a0, L)] for t in range(NT)]
            f1 = (v[0] * 3 + v[5]) * RSTRIDE
            f2 = (v[2] * 10 + v[4]) * RSTRIDE
            f3 = (v[1] * 35 + v[3] * 7 + v[6]) * RSTRIDE
            zero = jnp.zeros((L,), jnp.int32)
            f1 = lax.max(lax.min(f1, (G1_ROWS - 1) * RSTRIDE), zero)
            f2 = lax.max(lax.min(f2, (G2_ROWS - 1) * RSTRIDE), zero)
            f3 = lax.max(lax.min(f3, (G3_ROWS - 1) * RSTRIDE), zero)
            for k in range(L):
                r1 = plsc.bitcast(
                    plsc.load_gather(g1_v, [f1 + k]), jnp.bfloat16)
                r2 = plsc.bitcast(
                    plsc.load_gather(g2_v, [f2 + k]), jnp.bfloat16)
                r3 = plsc.bitcast(
                    plsc.load_gather(g3_v, [f3 + k]), jnp.bfloat16)
                acc_k, acc_khi = plsc.unpack(
                    r1 + r2 + r3, format=plsc.PackFormat.INTERLEAVED)
                out_v[cur][pl.ds(k * CHUNK + a0, L)] = acc_k
                out_v[cur][pl.ds((k + L) * CHUNK + a0, L)] = acc_khi

        out_inflight[ci] = fire_out(ci, cur)

    for cps in out_inflight.values():
        for cp in cps:
            cp.wait()


@jax.jit
def _run(idxs, tables_flat):
    mesh = plsc.VectorSubcoreMesh(
        core_axis_name="c", subcore_axis_name="s",
        num_cores=NC, num_subcores=NS)
    scratch = (
        [pltpu.VMEM((SIZES[t] * D,), jnp.float32) for t in range(NT)]
        + [pltpu.VMEM((G1_ROWS * RSTRIDE,), jnp.int32),
           pltpu.VMEM((G2_ROWS * RSTRIDE,), jnp.int32),
           pltpu.VMEM((U_ROWS * D,), jnp.float32),
           pltpu.VMEM((G3_ROWS * RSTRIDE,), jnp.int32)]
        + [pltpu.VMEM((CHUNK,), jnp.int32) for _ in range(2 * NT)]
        + [pltpu.VMEM((CHUNK * D,), jnp.float32) for _ in range(2)]
        + [pltpu.SemaphoreType.DMA for _ in range(4)]
    )
    fn = pl.kernel(
        _sc_body,
        out_type=jax.ShapeDtypeStruct((D, N), jnp.float32),
        mesh=mesh,
        scratch_types=scratch,
        compiler_params=pltpu.CompilerParams(needs_layout_passes=False),
    )
    return fn(*idxs, *tables_flat)


def kernel(atomic_num, formal_charge, degree, chiral_tag, total_numHs,
           is_aromatic, hybridization, W_atomic_num, W_formal_charge,
           W_degree, W_chiral_tag, W_total_numHs, W_is_aromatic,
           W_hybridization):
    idxs = [atomic_num, formal_charge, degree, chiral_tag, total_numHs,
            is_aromatic, hybridization]
    tables = [W_atomic_num, W_formal_charge, W_degree, W_chiral_tag,
              W_total_numHs, W_is_aromatic, W_hybridization]
    tables_flat = [w.reshape(-1) for w in tables]
    out = _run(idxs, tables_flat)
    return out.T


# speedup vs baseline: 5.7323x; 1.0268x over previous
import jax
import jax.numpy as jnp
from jax import lax
from jax.experimental import pallas as pl
from jax.experimental.pallas import tpu as pltpu
from jax.experimental.pallas import tpu_sc as plsc

N = 100000
D = 32
SIZES = (120, 17, 13, 5, 10, 3, 7)
NT = len(SIZES)

NC = 2
NS = 16
NW = NC * NS
L = 16

PER_TILE = 3200
CHUNK = 640
NCHUNKS = PER_TILE // CHUNK
LAST_START = 100096 - CHUNK

RSTRIDE = 17
G1_ROWS = 120 * 3
G2_ROWS = 13 * 10
U_ROWS = 17 * 5
G3_ROWS = U_ROWS * 7


def _row(ref, r):
    return (ref[pl.ds(r * D, L)], ref[pl.ds(r * D + L, L)])


def _sc_body(*refs):
    idx_hbm = refs[0:NT]
    w_hbm = refs[NT:2 * NT]
    out_hbm = refs[2 * NT]
    sc = refs[2 * NT + 1:]
    w_v = sc[0:NT]
    g1_v, g2_v, u_v, g3_v = sc[NT:NT + 4]
    idx_v = (sc[NT + 4:2 * NT + 4], sc[2 * NT + 4:3 * NT + 4])
    out_v = sc[3 * NT + 4:3 * NT + 6]
    idx_sem = sc[3 * NT + 6:3 * NT + 8]
    out_sem = sc[3 * NT + 8:3 * NT + 10]

    c = lax.axis_index("c")
    s = lax.axis_index("s")
    wid = s * NC + c
    base = wid * PER_TILE

    iota = lax.iota(jnp.int32, L)

    for t in range(NT):
        pltpu.sync_copy(w_hbm[t], w_v[t])

    def _pack_row(lo, hi):
        return plsc.bitcast(
            plsc.pack(lo, hi, format=plsc.PackFormat.INTERLEAVED), jnp.int32)

    def _combine_packed(dst, a_ref, b_rows_static, nb):
        def body(i, carry):
            alo, ahi = _row(a_ref, i)
            for r in range(nb):
                row = (i * nb + r) * RSTRIDE
                plsc.store_scatter(
                    dst, [iota + row],
                    _pack_row(alo + b_rows_static[r][0],
                              ahi + b_rows_static[r][1]))
            return carry
        return body

    def _combine_f32(dst, a_ref, b_rows_static, nb):
        def body(i, carry):
            alo, ahi = _row(a_ref, i)
            for r in range(nb):
                row = (i * nb + r) * D
                dst[pl.ds(row, L)] = alo + b_rows_static[r][0]
                dst[pl.ds(row + L, L)] = ahi + b_rows_static[r][1]
            return carry
        return body

    ar_rows = [_row(w_v[5], r) for r in range(3)]
    lax.fori_loop(0, 120, _combine_packed(g1_v, w_v[0], ar_rows, 3), 0)
    nh_rows = [_row(w_v[4], r) for r in range(10)]
    lax.fori_loop(0, 13, _combine_packed(g2_v, w_v[2], nh_rows, 10), 0)
    ct_rows = [_row(w_v[3], r) for r in range(5)]
    lax.fori_loop(0, 17, _combine_f32(u_v, w_v[1], ct_rows, 5), 0)
    hy_rows = [_row(w_v[6], r) for r in range(7)]
    lax.fori_loop(0, U_ROWS, _combine_packed(g3_v, u_v, hy_rows, 7), 0)

    def chunk_start(ci):
        return pl.multiple_of(jnp.minimum(base + ci * CHUNK, LAST_START), 8)

    def fire_idx(ci, buf):
        row0 = chunk_start(ci)
        return [
            pltpu.async_copy(
                idx_hbm[t].at[pl.ds(row0, CHUNK)], idx_v[buf][t], idx_sem[buf])
            for t in range(NT)
        ]

    def fire_out(ci, buf):
        row0 = chunk_start(ci)
        return [
            pltpu.async_copy(
                out_v[buf].at[pl.ds(d * CHUNK, CHUNK)],
                out_hbm.at[d, pl.ds(row0, CHUNK)], out_sem[buf])
            for d in range(D)
        ]

    idx_inflight = {0: fire_idx(0, 0)}
    out_inflight = {}

    for ci in range(NCHUNKS):
        cur = ci % 2
        if ci + 1 < NCHUNKS:
            idx_inflight[ci + 1] = fire_idx(ci + 1, 1 - cur)
        for cp in idx_inflight.pop(ci):
            cp.wait()
        if ci - 2 in out_inflight:
            for cp in out_inflight.pop(ci - 2):
                cp.wait()

        @plsc.parallel_loop(0, CHUNK // L, 1, unroll=4)
        def group_body(g, cur=cur):
            a0 = g * L
            v = [idx_v[cur][t][pl.ds(a0, L)] for t in range(NT)]
            f1 = (v[0] * 3 + v[5]) * RSTRIDE
            f2 = (v[2] * 10 + v[4]) * RSTRIDE
            f3 = (v[1] * 35 + v[3] * 7 + v[6]) * RSTRIDE
            zero = jnp.zeros((L,), jnp.int32)
            f1 = lax.max(lax.min(f1, (G1_ROWS - 1) * RSTRIDE), zero)
            f2 = lax.max(lax.min(f2, (G2_ROWS - 1) * RSTRIDE), zero)
            f3 = lax.max(lax.min(f3, (G3_ROWS - 1) * RSTRIDE), zero)
            for k in range(L):
                r1 = plsc.bitcast(
                    plsc.load_gather(g1_v, [f1 + k]), jnp.bfloat16)
                r2 = plsc.bitcast(
                    plsc.load_gather(g2_v, [f2 + k]), jnp.bfloat16)
                r3 = plsc.bitcast(
                    plsc.load_gather(g3_v, [f3 + k]), jnp.bfloat16)
                acc_k, acc_khi = plsc.unpack(
                    r1 + r2 + r3, format=plsc.PackFormat.INTERLEAVED)
                out_v[cur][pl.ds(k * CHUNK + a0, L)] = acc_k
                out_v[cur][pl.ds((k + L) * CHUNK + a0, L)] = acc_khi

        out_inflight[ci] = fire_out(ci, cur)

    for cps in out_inflight.values():
        for cp in cps:
            cp.wait()


@jax.jit
def _run(idxs, tables_flat):
    mesh = plsc.VectorSubcoreMesh(
        core_axis_name="c", subcore_axis_name="s",
        num_cores=NC, num_subcores=NS)
    scratch = (
        [pltpu.VMEM((SIZES[t] * D,), jnp.float32) for t in range(NT)]
        + [pltpu.VMEM((G1_ROWS * RSTRIDE,), jnp.int32),
           pltpu.VMEM((G2_ROWS * RSTRIDE,), jnp.int32),
           pltpu.VMEM((U_ROWS * D,), jnp.float32),
           pltpu.VMEM((G3_ROWS * RSTRIDE,), jnp.int32)]
        + [pltpu.VMEM((CHUNK,), jnp.int32) for _ in range(2 * NT)]
        + [pltpu.VMEM((CHUNK * D,), jnp.float32) for _ in range(2)]
        + [pltpu.SemaphoreType.DMA for _ in range(4)]
    )
    fn = pl.kernel(
        _sc_body,
        out_type=jax.ShapeDtypeStruct((D, N), jnp.float32),
        mesh=mesh,
        scratch_types=scratch,
        compiler_params=pltpu.CompilerParams(needs_layout_passes=False),
    )
    return fn(*idxs, *tables_flat)


def kernel(atomic_num, formal_charge, degree, chiral_tag, total_numHs,
           is_aromatic, hybridization, W_atomic_num, W_formal_charge,
           W_degree, W_chiral_tag, W_total_numHs, W_is_aromatic,
           W_hybridization):
    idxs = [atomic_num, formal_charge, degree, chiral_tag, total_numHs,
            is_aromatic, hybridization]
    tables = [W_atomic_num, W_formal_charge, W_degree, W_chiral_tag,
              W_total_numHs, W_is_aromatic, W_hybridization]
    tables_flat = [w.reshape(-1) for w in tables]
    out = _run(idxs, tables_flat)
    return out.T
